# Initial kernel scaffold; baseline (speedup 1.0000x reference)
import jax
import jax.numpy as jnp
from jax.experimental import pallas as pl


def kernel(idx_i, inter):
    raise NotImplementedError("write your pallas kernel here")


# speedup vs baseline: 5.7318x; 5.7318x over previous
import functools

import jax
import jax.numpy as jnp
from jax import lax
from jax.experimental import pallas as pl
from jax.experimental.pallas import tpu as pltpu
from jax.experimental.pallas import tpu_sc as plsc

N_EDGES = 320000
N_NODES = 10000
D = 128

NC, NS = 2, 16
NWORK = NC * NS
EPW = N_EDGES // NWORK
WIN = 200
NWIN = EPW // WIN
CH = 100
NCH = WIN // CH
IROWS = N_EDGES // CH
RPW = EPW // CH
IST = RPW + 4
RPS = 624
TAIL_R = N_NODES - NS * RPS


def _sc_partials(idx2d, inter):
    mesh = plsc.VectorSubcoreMesh(
        core_axis_name="c", subcore_axis_name="s",
        num_cores=NC, num_subcores=NS,
    )

    @functools.partial(
        pl.kernel,
        out_type=jax.ShapeDtypeStruct((NC, N_NODES, D), jnp.float32),
        mesh=mesh,
        scratch_types=[
            pltpu.VMEM((WIN, D), jnp.float32),
            pltpu.VMEM((IST, CH), jnp.int32),
            pltpu.VMEM_SHARED((N_NODES, D), jnp.float32),
        ],
    )
    def k(idx_hbm, inter_hbm, part_hbm, dbuf, ibuf, acc):
        cid = lax.axis_index("c")
        sid = lax.axis_index("s")
        wid = sid * NC + cid

        @pl.loop(0, WIN)
        def _(r):
            @pl.loop(0, D, step=16)
            def _(j):
                dbuf[r, pl.ds(j, 16)] = jnp.zeros((16,), jnp.float32)

        base_r = sid * RPS
        for t in range(3):
            pltpu.sync_copy(dbuf, acc.at[pl.ds(base_r + t * WIN, WIN)])
        pltpu.sync_copy(dbuf.at[pl.ds(0, RPS - 3 * WIN)],
                        acc.at[pl.ds(base_r + 3 * WIN, RPS - 3 * WIN)])

        @pl.when(sid == NS - 1)
        def _():
            pltpu.sync_copy(dbuf.at[pl.ds(0, TAIL_R)],
                            acc.at[pl.ds(NS * RPS, TAIL_R)])

        plsc.subcore_barrier()

        r0 = lax.rem(wid * RPW, 8)
        arow = pl.multiple_of(wid * RPW - r0, 8)
        pltpu.sync_copy(idx_hbm.at[pl.ds(arow, IST)], ibuf)

        ebase = wid * EPW

        @pl.loop(0, NWIN)
        def _(w):
            pltpu.sync_copy(inter_hbm.at[pl.ds(ebase + w * WIN, WIN)], dbuf)
            for c in range(NCH):
                pltpu.sync_copy(dbuf.at[pl.ds(c * CH, CH)],
                                acc.at[ibuf.at[r0 + w * NCH + c]], add=True)

        plsc.subcore_barrier()
        pltpu.sync_copy(acc.at[pl.ds(base_r, RPS)],
                        part_hbm.at[cid, pl.ds(base_r, RPS)])

        @pl.when(sid == NS - 1)
        def _():
            pltpu.sync_copy(acc.at[pl.ds(NS * RPS, TAIL_R)],
                            part_hbm.at[cid, pl.ds(NS * RPS, TAIL_R)])

    return k(idx2d, inter)


_CBLK = 2000


def _combine(parts):
    def body(p_ref, o_ref):
        o_ref[...] = p_ref[0] + p_ref[1]

    return pl.pallas_call(
        body,
        grid=(N_NODES // _CBLK,),
        in_specs=[pl.BlockSpec((NC, _CBLK, D), lambda i: (0, i, 0))],
        out_specs=pl.BlockSpec((_CBLK, D), lambda i: (i, 0)),
        out_shape=jax.ShapeDtypeStruct((N_NODES, D), jnp.float32),
    )(parts)


def kernel(idx_i, inter):
    idx2d = idx_i.astype(jnp.int32).reshape(IROWS, CH)
    parts = _sc_partials(idx2d, inter)
    return _combine(parts)


# speedup vs baseline: 7.4241x; 1.2952x over previous
import functools

import jax
import jax.numpy as jnp
from jax import lax
from jax.experimental import pallas as pl
from jax.experimental.pallas import tpu as pltpu
from jax.experimental.pallas import tpu_sc as plsc

N_EDGES = 320000
N_NODES = 10000
D = 128

NC, NS = 2, 16
NWORK = NC * NS
EPW = N_EDGES // NWORK
CH = 80
KPW = EPW // CH
IROWS = N_EDGES // CH
RPW = EPW // CH
IST = RPW + 11
IROWS_P = IROWS + 16
RPS = 624
TAIL_R = N_NODES - NS * RPS


def _sc_partials(idx2d, inter):
    mesh = plsc.VectorSubcoreMesh(
        core_axis_name="c", subcore_axis_name="s",
        num_cores=NC, num_subcores=NS,
    )

    @functools.partial(
        pl.kernel,
        out_type=jax.ShapeDtypeStruct((NC, N_NODES, D), jnp.float32),
        mesh=mesh,
        scratch_types=[
            pltpu.VMEM((CH, D), jnp.float32),
            pltpu.VMEM((CH, D), jnp.float32),
            pltpu.VMEM((IST, CH), jnp.int32),
            pltpu.VMEM_SHARED((N_NODES, D), jnp.float32),
            pltpu.SemaphoreType.DMA,
            pltpu.SemaphoreType.DMA,
            pltpu.SemaphoreType.DMA,
            pltpu.SemaphoreType.DMA,
        ],
    )
    def k(idx_hbm, inter_hbm, part_hbm, bufa, bufb, ibuf, acc,
          gsa, gsb, ssa, ssb):
        cid = lax.axis_index("c")
        sid = lax.axis_index("s")
        wid = sid * NC + cid

        @pl.loop(0, CH)
        def _(r):
            @pl.loop(0, D, step=16)
            def _(j):
                bufa[r, pl.ds(j, 16)] = jnp.zeros((16,), jnp.float32)

        base_r = sid * RPS
        @pl.loop(0, RPS // CH)
        def _(t):
            pltpu.sync_copy(bufa, acc.at[pl.ds(base_r + t * CH, CH)])
        pltpu.sync_copy(bufa.at[pl.ds(0, RPS % CH)],
                        acc.at[pl.ds(base_r + (RPS // CH) * CH, RPS % CH)])

        @pl.when(sid == NS - 1)
        def _():
            pltpu.sync_copy(bufa.at[pl.ds(0, TAIL_R)],
                            acc.at[pl.ds(NS * RPS, TAIL_R)])

        plsc.subcore_barrier()

        r0 = lax.rem(wid * RPW, 8)
        arow = pl.multiple_of(wid * RPW - r0, 8)
        pltpu.sync_copy(idx_hbm.at[pl.ds(arow, IST)], ibuf)

        ebase = wid * EPW

        def g_desc(c, buf, sem):
            return pltpu.make_async_copy(
                inter_hbm.at[pl.ds(ebase + c * CH, CH)], buf, sem)

        def s_desc(c, buf, sem):
            return pltpu.make_async_copy(buf, acc.at[ibuf.at[r0 + c]], sem)

        g_desc(0, bufa, gsa).start()

        @pl.loop(0, (KPW - 1) // 2)
        def _(j):
            c = 2 * j
            g_desc(c + 1, bufb, gsb).start()
            g_desc(c, bufa, gsa).wait()
            s_desc(c, bufa, ssa).start(add=True)
            s_desc(c, bufa, ssa).wait()
            g_desc(c + 2, bufa, gsa).start()
            g_desc(c + 1, bufb, gsb).wait()
            s_desc(c + 1, bufb, ssb).start(add=True)
            s_desc(c + 1, bufb, ssb).wait()

        last = KPW - 1
        g_desc(last, bufa, gsa).wait()
        s_desc(last, bufa, ssa).start(add=True)
        s_desc(last, bufa, ssa).wait()

        plsc.subcore_barrier()
        pltpu.sync_copy(acc.at[pl.ds(base_r, RPS)],
                        part_hbm.at[cid, pl.ds(base_r, RPS)])

        @pl.when(sid == NS - 1)
        def _():
            pltpu.sync_copy(acc.at[pl.ds(NS * RPS, TAIL_R)],
                            part_hbm.at[cid, pl.ds(NS * RPS, TAIL_R)])

    return k(idx2d, inter)


_CBLK = 2000


def _combine(parts):
    def body(p_ref, o_ref):
        o_ref[...] = p_ref[0] + p_ref[1]

    return pl.pallas_call(
        body,
        grid=(N_NODES // _CBLK,),
        in_specs=[pl.BlockSpec((NC, _CBLK, D), lambda i: (0, i, 0))],
        out_specs=pl.BlockSpec((_CBLK, D), lambda i: (i, 0)),
        out_shape=jax.ShapeDtypeStruct((N_NODES, D), jnp.float32),
    )(parts)


def kernel(idx_i, inter):
    idx2d = idx_i.astype(jnp.int32).reshape(IROWS, CH)
    idx2d = jnp.pad(idx2d, ((0, IROWS_P - IROWS), (0, 0)))
    parts = _sc_partials(idx2d, inter)
    return _combine(parts)
